# baseline (speedup 1.0000x reference)
import functools

import jax
import jax.numpy as jnp
from jax import lax
from jax.experimental import pallas as pl
from jax.experimental.pallas import tpu as pltpu
from jax.experimental.pallas import tpu_sc as plsc

NC, NS = 2, 16
NW = NC * NS
BATCH, HIST, EMB = 16384, 200, 32
B = BATCH * HIST
VOCAB = 1048576

CT, RT, CL, RL = 4, 8192, 8, 128
RT_PER_W = RT // NW
HT, BT, HL, BL = 25, 128, 8, 128
NTILE = HT * BT
TILE_PER_W = NTILE // NW

_mesh = plsc.VectorSubcoreMesh(core_axis_name="c", subcore_axis_name="s")


def _wid():
    return lax.axis_index("s") * NC + lax.axis_index("c")


@functools.partial(
    pl.kernel,
    out_type=jax.ShapeDtypeStruct((VOCAB * EMB,), jnp.float32),
    mesh=_mesh,
    scratch_types=[
        pltpu.VMEM((CT * 1024,), jnp.float32),
        pltpu.VMEM((CT * 1024,), jnp.float32),
        pltpu.VMEM((4096,), jnp.float32),
        pltpu.VMEM((4096,), jnp.float32),
        pltpu.SemaphoreType.DMA((2,)),
        pltpu.SemaphoreType.DMA((2,)),
    ],
    compiler_params=pltpu.CompilerParams(use_tc_tiling_on_sc=False, needs_layout_passes=False, disable_bounds_checks=True),
)
def _retile_kernel(tphys, trm, b1a, b1b, b2a, b2b, semi, semo):
    w = _wid()
    iota = lax.iota(jnp.int32, 16)
    b1 = (b1a, b1b)
    b2 = (b2a, b2b)

    def in_dmas(k, par):
        rt = w * RT_PER_W + k
        return [
            pltpu.make_async_copy(
                tphys.at[pl.ds((ct * RT + rt) * 1024, 1024)],
                b1[par].at[pl.ds(ct * 1024, 1024)],
                semi.at[par],
            )
            for ct in range(CT)
        ]

    def out_dma(k, par):
        rt = w * RT_PER_W + k
        return pltpu.make_async_copy(
            b2[par], trm.at[pl.ds(rt * 4096, 4096)], semo.at[par]
        )

    for d in in_dmas(0, 0):
        d.start()
    for d in in_dmas(1, 1):
        d.start()

    def pair(k2, _):
        for par in range(2):
            k = k2 * 2 + par
            for d in in_dmas(k, par):
                d.wait()

            @pl.when(k2 >= 1)
            def _():
                out_dma(k - 2, par).wait()

            src, dst = b1[par], b2[par]

            def diag(dk, _):
                csk = jnp.bitwise_and(iota + dk, 15)
                psrc = csk * 128 + iota
                qdst = iota * 32 + csk
                for c0 in (0, 16):
                    for r8 in range(8):
                        rl0 = r8 * 16
                        val = plsc.load_gather(
                            src, [psrc + (c0 * 128 + rl0)]
                        )
                        plsc.store_scatter(
                            dst, [qdst + (rl0 * 32 + c0)], val
                        )
                return ()

            lax.fori_loop(0, 16, diag, ())
            out_dma(k, par).start()

            @pl.when(k2 < RT_PER_W // 2 - 1)
            def _():
                for d in in_dmas(k + 2, par):
                    d.start()
        return ()

    lax.fori_loop(0, RT_PER_W // 2, pair, ())
    for par in range(2):
        out_dma(RT_PER_W - 2 + par, par).wait()


@functools.partial(
    pl.kernel,
    out_type=jax.ShapeDtypeStruct((B * EMB,), jnp.float32),
    mesh=_mesh,
    scratch_types=[
        pltpu.VMEM((1024,), jnp.int32),
        pltpu.VMEM((1024,), jnp.int32),
        pltpu.VMEM((1024, EMB), jnp.float32),
        pltpu.VMEM((1024, EMB), jnp.float32),
        pltpu.VMEM((2056,), jnp.float32),
        pltpu.VMEM((2056,), jnp.float32),
        pltpu.VMEM((2056,), jnp.float32),
        pltpu.VMEM((2056,), jnp.float32),
        pltpu.VMEM((2056,), jnp.float32),
        pltpu.VMEM((2056,), jnp.float32),
        pltpu.VMEM((2056,), jnp.float32),
        pltpu.VMEM((2056,), jnp.float32),
        pltpu.SemaphoreType.DMA((2,)),
        pltpu.SemaphoreType.DMA((2,)),
        pltpu.SemaphoreType.DMA((8,)),
    ],
    compiler_params=pltpu.CompilerParams(use_tc_tiling_on_sc=False, needs_layout_passes=False, disable_bounds_checks=True),
)
def _gather_kernel(cphys, trm, outh, idxa, idxb, ga, gbuf,
                   t0, t1, t2, t3, t4, t5, t6, t7, semi, semg, semo):
    tt = (t0, t1, t2, t3, t4, t5, t6, t7)
    w = _wid()
    iota = lax.iota(jnp.int32, 16)
    idxs = (idxa, idxb)
    gs = (ga, gbuf)

    def idx_dma(p, par):
        tile = w * TILE_PER_W + p
        return pltpu.make_async_copy(
            cphys.at[pl.ds(tile * 1024, 1024)], idxs[par], semi.at[par]
        )

    def gather_dma(par):
        return pltpu.make_async_copy(trm.at[idxs[par]], gs[par], semg.at[par])

    idx_dma(0, 0).start()
    idx_dma(0, 0).wait()
    gather_dma(0).start()
    idx_dma(1, 1).start()

    i7 = jnp.bitwise_and(iota, 7)
    h8 = lax.shift_right_logical(iota, 3) * 8
    hskew = lax.shift_right_logical(iota, 3) * 1032 + i7

    def emit(p, par):
        tile = w * TILE_PER_W + p
        ht = tile // BT
        bt = tile - ht * BT
        src = gs[par]

        def hl_body(hl2, _):
            for hlq in range(4):
                hl = hl2 * 4 + hlq
                for ehp in range(2):
                    slot = hlq * 2 + ehp

                    @pl.when((p > 0) | (hl2 > 0))
                    def _():
                        for _h in range(2):
                            pltpu.make_async_copy(
                                tt[slot].at[pl.ds(0, 1024)],
                                outh.at[pl.ds(0, 1024)],
                                semo.at[slot],
                            ).wait()

                    dst = tt[slot]
                    rows_b = [i7 + (hl * 128 + b16 * 8) for b16 in range(16)]
                    cols_base = h8 + ehp * 16

                    def dk_body(dkh, _):
                        for j in range(4):
                            csk = jnp.bitwise_and(i7 + (dkh * 4 + j), 7)
                            cols = cols_base + csk
                            wdst = hskew + csk * 128
                            for b16 in range(16):
                                val = plsc.load_gather(
                                    src, [rows_b[b16], cols]
                                )
                                plsc.store_scatter(dst, [wdst + b16 * 8], val)
                        return ()

                    lax.fori_loop(0, 2, dk_body, ())
                    for h in range(2):
                        off = (
                            ((ht * 8 + hl) * 4 + ehp * 2 + h) * 128 + bt
                        ) * 1024
                        pltpu.async_copy(
                            dst.at[pl.ds(h * 1032, 1024)],
                            outh.at[pl.ds(off, 1024)],
                            semo.at[slot],
                        )
            return ()

        lax.fori_loop(0, 2, hl_body, ())

    def pair(p2, _):
        more = p2 < TILE_PER_W // 2 - 1
        p = 2 * p2

        idx_dma(p + 1, 1).wait()
        gather_dma(1).start()
        gather_dma(0).wait()

        @pl.when(more)
        def _():
            idx_dma(p + 2, 0).start()

        emit(p, 0)

        @pl.when(more)
        def _():
            idx_dma(p + 2, 0).wait()
            gather_dma(0).start()

        gather_dma(1).wait()

        @pl.when(more)
        def _():
            idx_dma(p + 3, 1).start()

        emit(p + 1, 1)
        return ()

    lax.fori_loop(0, TILE_PER_W // 2, pair, ())
    for slot in range(8):
        for _h in range(2):
            pltpu.make_async_copy(
                tt[slot].at[pl.ds(0, 1024)], outh.at[pl.ds(0, 1024)],
                semo.at[slot],
            ).wait()


@jax.jit
def kernel(condition, table):
    cphys = jnp.transpose(
        condition.reshape(BT, BL, HT, HL), (2, 0, 3, 1)
    ).reshape(-1)
    tphys = jnp.transpose(
        table.reshape(RT, RL, CT, CL), (2, 0, 3, 1)
    ).reshape(-1)
    trm = _retile_kernel(tphys).reshape(VOCAB, EMB)
    outflat = _gather_kernel(cphys, trm)
    out = jnp.transpose(
        outflat.reshape(HIST, 4, BT, 8, BL), (2, 4, 0, 1, 3)
    ).reshape(BATCH, HIST, EMB)
    return out


# speedup vs baseline: 1.2824x; 1.2824x over previous
import functools

import jax
import jax.numpy as jnp
from jax import lax
from jax.experimental import pallas as pl
from jax.experimental.pallas import tpu as pltpu
from jax.experimental.pallas import tpu_sc as plsc

NC, NS = 2, 16
NW = NC * NS
BATCH, HIST, EMB = 16384, 200, 32
B = BATCH * HIST
VOCAB = 1048576

CT, RT, CL, RL = 4, 8192, 8, 128
RT_PER_W = RT // NW
HT, BT, HL, BL = 25, 128, 8, 128
NTILE = HT * BT
TILE_PER_W = NTILE // NW

_mesh = plsc.VectorSubcoreMesh(core_axis_name="c", subcore_axis_name="s")


def _wid():
    return lax.axis_index("s") * NC + lax.axis_index("c")


@functools.partial(
    pl.kernel,
    out_type=jax.ShapeDtypeStruct((VOCAB * EMB,), jnp.float32),
    mesh=_mesh,
    scratch_types=[
        pltpu.VMEM((CT * 1024,), jnp.float32),
        pltpu.VMEM((CT * 1024,), jnp.float32),
        pltpu.VMEM((4096,), jnp.float32),
        pltpu.VMEM((4096,), jnp.float32),
        pltpu.SemaphoreType.DMA((2,)),
        pltpu.SemaphoreType.DMA((2,)),
    ],
    compiler_params=pltpu.CompilerParams(use_tc_tiling_on_sc=False, needs_layout_passes=False, disable_bounds_checks=True),
)
def _retile_kernel(tphys, trm, b1a, b1b, b2a, b2b, semi, semo):
    w = _wid()
    iota = lax.iota(jnp.int32, 16)
    b1 = (b1a, b1b)
    b2 = (b2a, b2b)

    def in_dmas(k, par):
        rt = w * RT_PER_W + k
        return [
            pltpu.make_async_copy(
                tphys.at[pl.ds((ct * RT + rt) * 1024, 1024)],
                b1[par].at[pl.ds(ct * 1024, 1024)],
                semi.at[par],
            )
            for ct in range(CT)
        ]

    def out_dma(k, par):
        rt = w * RT_PER_W + k
        return pltpu.make_async_copy(
            b2[par], trm.at[pl.ds(rt * 4096, 4096)], semo.at[par]
        )

    for d in in_dmas(0, 0):
        d.start()
    for d in in_dmas(1, 1):
        d.start()

    def pair(k2, _):
        for par in range(2):
            k = k2 * 2 + par
            for d in in_dmas(k, par):
                d.wait()

            @pl.when(k2 >= 1)
            def _():
                out_dma(k - 2, par).wait()

            src, dst = b1[par], b2[par]

            def diag(dk, _):
                csk = jnp.bitwise_and(iota + dk, 15)
                psrc = csk * 128 + iota
                qdst = iota * 32 + csk
                for c0 in (0, 16):
                    for r8 in range(8):
                        rl0 = r8 * 16
                        val = plsc.load_gather(
                            src, [psrc + (c0 * 128 + rl0)]
                        )
                        plsc.store_scatter(
                            dst, [qdst + (rl0 * 32 + c0)], val
                        )
                return ()

            lax.fori_loop(0, 16, diag, ())
            out_dma(k, par).start()

            @pl.when(k2 < RT_PER_W // 2 - 1)
            def _():
                for d in in_dmas(k + 2, par):
                    d.start()
        return ()

    lax.fori_loop(0, RT_PER_W // 2, pair, ())
    for par in range(2):
        out_dma(RT_PER_W - 2 + par, par).wait()


@functools.partial(
    pl.kernel,
    out_type=jax.ShapeDtypeStruct((B * EMB,), jnp.float32),
    mesh=_mesh,
    scratch_types=[
        pltpu.VMEM((1024,), jnp.int32),
        pltpu.VMEM((1024,), jnp.int32),
        pltpu.VMEM((1024, EMB), jnp.float32),
        pltpu.VMEM((1024, EMB), jnp.float32),
        pltpu.VMEM((2056,), jnp.float32),
        pltpu.VMEM((2056,), jnp.float32),
        pltpu.VMEM((2056,), jnp.float32),
        pltpu.VMEM((2056,), jnp.float32),
        pltpu.VMEM((2056,), jnp.float32),
        pltpu.VMEM((2056,), jnp.float32),
        pltpu.VMEM((2056,), jnp.float32),
        pltpu.VMEM((2056,), jnp.float32),
        pltpu.SemaphoreType.DMA((2,)),
        pltpu.SemaphoreType.DMA((2,)),
        pltpu.SemaphoreType.DMA((8,)),
    ],
    compiler_params=pltpu.CompilerParams(use_tc_tiling_on_sc=False, needs_layout_passes=False, disable_bounds_checks=True),
)
def _gather_kernel(cphys, trm, outh, idxa, idxb, ga, gbuf,
                   t0, t1, t2, t3, t4, t5, t6, t7, semi, semg, semo):
    tt = (t0, t1, t2, t3, t4, t5, t6, t7)
    w = _wid()
    iota = lax.iota(jnp.int32, 16)
    idxs = (idxa, idxb)
    gs = (ga, gbuf)

    def idx_dma(p, par):
        tile = w * TILE_PER_W + p
        return pltpu.make_async_copy(
            cphys.at[pl.ds(tile * 1024, 1024)], idxs[par], semi.at[par]
        )

    def gather_dma(par):
        return pltpu.make_async_copy(trm.at[idxs[par]], gs[par], semg.at[par])

    idx_dma(0, 0).start()
    idx_dma(0, 0).wait()
    gather_dma(0).start()
    idx_dma(1, 1).start()

    i7 = jnp.bitwise_and(iota, 7)
    h8 = lax.shift_right_logical(iota, 3) * 8
    hskew = lax.shift_right_logical(iota, 3) * 1032 + i7

    def emit(p, par):
        tile = w * TILE_PER_W + p
        ht = tile // BT
        bt = tile - ht * BT
        src = gs[par]

        def hl_body(hl2, _):
            for hlq in range(4):
                hl = hl2 * 4 + hlq
                for ehp in range(2):
                    slot = hlq * 2 + ehp

                    @pl.when((p > 0) | (hl2 > 0))
                    def _():
                        for _h in range(2):
                            pltpu.make_async_copy(
                                tt[slot].at[pl.ds(0, 1024)],
                                outh.at[pl.ds(0, 1024)],
                                semo.at[slot],
                            ).wait()

                    dst = tt[slot]

                    @plsc.parallel_loop(0, 8, 1, unroll=1)
                    def _(dk):
                        csk = jnp.bitwise_and(i7 + dk, 7)
                        cols = h8 + csk + ehp * 16
                        wdst = hskew + csk * 128
                        for b16 in range(16):
                            rows = i7 + (hl * 128 + b16 * 8)
                            val = plsc.load_gather(src, [rows, cols])
                            plsc.store_scatter(dst, [wdst + b16 * 8], val)
                    for h in range(2):
                        off = (
                            ((ht * 8 + hl) * 4 + ehp * 2 + h) * 128 + bt
                        ) * 1024
                        pltpu.async_copy(
                            dst.at[pl.ds(h * 1032, 1024)],
                            outh.at[pl.ds(off, 1024)],
                            semo.at[slot],
                        )
            return ()

        lax.fori_loop(0, 2, hl_body, ())

    def pair(p2, _):
        more = p2 < TILE_PER_W // 2 - 1
        p = 2 * p2

        idx_dma(p + 1, 1).wait()
        gather_dma(1).start()
        gather_dma(0).wait()

        @pl.when(more)
        def _():
            idx_dma(p + 2, 0).start()

        emit(p, 0)

        @pl.when(more)
        def _():
            idx_dma(p + 2, 0).wait()
            gather_dma(0).start()

        gather_dma(1).wait()

        @pl.when(more)
        def _():
            idx_dma(p + 3, 1).start()

        emit(p + 1, 1)
        return ()

    lax.fori_loop(0, TILE_PER_W // 2, pair, ())
    for slot in range(8):
        for _h in range(2):
            pltpu.make_async_copy(
                tt[slot].at[pl.ds(0, 1024)], outh.at[pl.ds(0, 1024)],
                semo.at[slot],
            ).wait()


@jax.jit
def kernel(condition, table):
    cphys = jnp.transpose(
        condition.reshape(BT, BL, HT, HL), (2, 0, 3, 1)
    ).reshape(-1)
    tphys = jnp.transpose(
        table.reshape(RT, RL, CT, CL), (2, 0, 3, 1)
    ).reshape(-1)
    trm = _retile_kernel(tphys).reshape(VOCAB, EMB)
    outflat = _gather_kernel(cphys, trm)
    out = jnp.transpose(
        outflat.reshape(HIST, 4, BT, 8, BL), (2, 4, 0, 1, 3)
    ).reshape(BATCH, HIST, EMB)
    return out


# speedup vs baseline: 1.4124x; 1.1014x over previous
import functools

import jax
import jax.numpy as jnp
from jax import lax
from jax.experimental import pallas as pl
from jax.experimental.pallas import tpu as pltpu
from jax.experimental.pallas import tpu_sc as plsc

NC, NS = 2, 16
NW = NC * NS
BATCH, HIST, EMB = 16384, 200, 32
B = BATCH * HIST
VOCAB = 1048576

CT, RT, CL, RL = 4, 8192, 8, 128
RT_PER_W = RT // NW
HT, BT, HL, BL = 25, 128, 8, 128
NTILE = HT * BT
TILE_PER_W = NTILE // NW

_mesh = plsc.VectorSubcoreMesh(core_axis_name="c", subcore_axis_name="s")


def _wid():
    return lax.axis_index("s") * NC + lax.axis_index("c")


@functools.partial(
    pl.kernel,
    out_type=jax.ShapeDtypeStruct((VOCAB * EMB,), jnp.float32),
    mesh=_mesh,
    scratch_types=[
        pltpu.VMEM((CT * 1024,), jnp.float32),
        pltpu.VMEM((CT * 1024,), jnp.float32),
        pltpu.VMEM((4096,), jnp.float32),
        pltpu.VMEM((4096,), jnp.float32),
        pltpu.SemaphoreType.DMA((2,)),
        pltpu.SemaphoreType.DMA((2,)),
    ],
    compiler_params=pltpu.CompilerParams(use_tc_tiling_on_sc=False, needs_layout_passes=False, disable_bounds_checks=True),
)
def _retile_kernel(tphys, trm, b1a, b1b, b2a, b2b, semi, semo):
    w = _wid()
    iota = lax.iota(jnp.int32, 16)
    b1 = (b1a, b1b)
    b2 = (b2a, b2b)

    def in_dmas(k, par):
        rt = w * RT_PER_W + k
        return [
            pltpu.make_async_copy(
                tphys.at[pl.ds((ct * RT + rt) * 1024, 1024)],
                b1[par].at[pl.ds(ct * 1024, 1024)],
                semi.at[par],
            )
            for ct in range(CT)
        ]

    def out_dma(k, par):
        rt = w * RT_PER_W + k
        return pltpu.make_async_copy(
            b2[par], trm.at[pl.ds(rt * 4096, 4096)], semo.at[par]
        )

    for d in in_dmas(0, 0):
        d.start()
    for d in in_dmas(1, 1):
        d.start()

    def pair(k2, _):
        for par in range(2):
            k = k2 * 2 + par
            for d in in_dmas(k, par):
                d.wait()

            @pl.when(k2 >= 1)
            def _():
                out_dma(k - 2, par).wait()

            src, dst = b1[par], b2[par]

            @plsc.parallel_loop(0, 16, 1, unroll=1)
            def _(dk):
                csk = jnp.bitwise_and(iota + dk, 15)
                psrc = csk * 128 + iota
                qdst = iota * 32 + csk
                for c0 in (0, 16):
                    for r8 in range(8):
                        rl0 = r8 * 16
                        val = plsc.load_gather(
                            src, [psrc + (c0 * 128 + rl0)]
                        )
                        plsc.store_scatter(
                            dst, [qdst + (rl0 * 32 + c0)], val
                        )
            out_dma(k, par).start()

            @pl.when(k2 < RT_PER_W // 2 - 1)
            def _():
                for d in in_dmas(k + 2, par):
                    d.start()
        return ()

    lax.fori_loop(0, RT_PER_W // 2, pair, ())
    for par in range(2):
        out_dma(RT_PER_W - 2 + par, par).wait()


@functools.partial(
    pl.kernel,
    out_type=jax.ShapeDtypeStruct((B * EMB,), jnp.float32),
    mesh=_mesh,
    scratch_types=[
        pltpu.VMEM((1024,), jnp.int32),
        pltpu.VMEM((1024,), jnp.int32),
        pltpu.VMEM((1024, EMB), jnp.float32),
        pltpu.VMEM((1024, EMB), jnp.float32),
        pltpu.VMEM((2056,), jnp.float32),
        pltpu.VMEM((2056,), jnp.float32),
        pltpu.VMEM((2056,), jnp.float32),
        pltpu.VMEM((2056,), jnp.float32),
        pltpu.VMEM((2056,), jnp.float32),
        pltpu.VMEM((2056,), jnp.float32),
        pltpu.VMEM((2056,), jnp.float32),
        pltpu.VMEM((2056,), jnp.float32),
        pltpu.SemaphoreType.DMA((2,)),
        pltpu.SemaphoreType.DMA((2,)),
        pltpu.SemaphoreType.DMA((8,)),
    ],
    compiler_params=pltpu.CompilerParams(use_tc_tiling_on_sc=False, needs_layout_passes=False, disable_bounds_checks=True),
)
def _gather_kernel(cphys, trm, outh, idxa, idxb, ga, gbuf,
                   t0, t1, t2, t3, t4, t5, t6, t7, semi, semg, semo):
    tt = (t0, t1, t2, t3, t4, t5, t6, t7)
    w = _wid()
    iota = lax.iota(jnp.int32, 16)
    idxs = (idxa, idxb)
    gs = (ga, gbuf)

    def idx_dma(p, par):
        tile = w * TILE_PER_W + p
        return pltpu.make_async_copy(
            cphys.at[pl.ds(tile * 1024, 1024)], idxs[par], semi.at[par]
        )

    def gather_dma(par):
        return pltpu.make_async_copy(trm.at[idxs[par]], gs[par], semg.at[par])

    idx_dma(0, 0).start()
    idx_dma(0, 0).wait()
    gather_dma(0).start()
    idx_dma(1, 1).start()

    i7 = jnp.bitwise_and(iota, 7)
    h8 = lax.shift_right_logical(iota, 3) * 8
    hskew = lax.shift_right_logical(iota, 3) * 1032 + i7

    def emit(p, par):
        tile = w * TILE_PER_W + p
        ht = tile // BT
        bt = tile - ht * BT
        src = gs[par]

        def hl_body(hl2, _):
            for hlq in range(4):
                hl = hl2 * 4 + hlq
                for ehp in range(2):
                    slot = hlq * 2 + ehp

                    @pl.when((p > 0) | (hl2 > 0))
                    def _():
                        for _h in range(2):
                            pltpu.make_async_copy(
                                tt[slot].at[pl.ds(0, 1024)],
                                outh.at[pl.ds(0, 1024)],
                                semo.at[slot],
                            ).wait()

                    dst = tt[slot]

                    @plsc.parallel_loop(0, 8, 1, unroll=1)
                    def _(dk):
                        csk = jnp.bitwise_and(i7 + dk, 7)
                        cols = h8 + csk + ehp * 16
                        wdst = hskew + csk * 128
                        for b16 in range(16):
                            rows = i7 + (hl * 128 + b16 * 8)
                            val = plsc.load_gather(src, [rows, cols])
                            plsc.store_scatter(dst, [wdst + b16 * 8], val)
                    for h in range(2):
                        off = (
                            ((ht * 8 + hl) * 4 + ehp * 2 + h) * 128 + bt
                        ) * 1024
                        pltpu.async_copy(
                            dst.at[pl.ds(h * 1032, 1024)],
                            outh.at[pl.ds(off, 1024)],
                            semo.at[slot],
                        )
            return ()

        lax.fori_loop(0, 2, hl_body, ())

    def pair(p2, _):
        more = p2 < TILE_PER_W // 2 - 1
        p = 2 * p2

        idx_dma(p + 1, 1).wait()
        gather_dma(1).start()
        gather_dma(0).wait()

        @pl.when(more)
        def _():
            idx_dma(p + 2, 0).start()

        emit(p, 0)

        @pl.when(more)
        def _():
            idx_dma(p + 2, 0).wait()
            gather_dma(0).start()

        gather_dma(1).wait()

        @pl.when(more)
        def _():
            idx_dma(p + 3, 1).start()

        emit(p + 1, 1)
        return ()

    lax.fori_loop(0, TILE_PER_W // 2, pair, ())
    for slot in range(8):
        for _h in range(2):
            pltpu.make_async_copy(
                tt[slot].at[pl.ds(0, 1024)], outh.at[pl.ds(0, 1024)],
                semo.at[slot],
            ).wait()


@jax.jit
def kernel(condition, table):
    cphys = jnp.transpose(
        condition.reshape(BT, BL, HT, HL), (2, 0, 3, 1)
    ).reshape(-1)
    tphys = jnp.transpose(
        table.reshape(RT, RL, CT, CL), (2, 0, 3, 1)
    ).reshape(-1)
    trm = _retile_kernel(tphys).reshape(VOCAB, EMB)
    outflat = _gather_kernel(cphys, trm)
    out = jnp.transpose(
        outflat.reshape(HIST, 4, BT, 8, BL), (2, 4, 0, 1, 3)
    ).reshape(BATCH, HIST, EMB)
    return out


# speedup vs baseline: 1.5608x; 1.1051x over previous
import functools

import jax
import jax.numpy as jnp
from jax import lax
from jax.experimental import pallas as pl
from jax.experimental.pallas import tpu as pltpu
from jax.experimental.pallas import tpu_sc as plsc

NC, NS = 2, 16
NW = NC * NS
BATCH, HIST, EMB = 16384, 200, 32
B = BATCH * HIST
VOCAB = 1048576

CT, RT, CL, RL = 4, 8192, 8, 128
RT_PER_W = RT // NW
HT, BT, HL, BL = 25, 128, 8, 128
NTILE = HT * BT
TILE_PER_W = NTILE // NW

_mesh = plsc.VectorSubcoreMesh(core_axis_name="c", subcore_axis_name="s")


def _wid():
    return lax.axis_index("s") * NC + lax.axis_index("c")


@functools.partial(
    pl.kernel,
    out_type=jax.ShapeDtypeStruct((VOCAB * EMB,), jnp.float32),
    mesh=_mesh,
    scratch_types=[
        pltpu.VMEM((CT * 1024,), jnp.float32),
        pltpu.VMEM((CT * 1024,), jnp.float32),
        pltpu.VMEM((4096,), jnp.float32),
        pltpu.VMEM((4096,), jnp.float32),
        pltpu.SemaphoreType.DMA((2,)),
        pltpu.SemaphoreType.DMA((2,)),
    ],
    compiler_params=pltpu.CompilerParams(use_tc_tiling_on_sc=False, needs_layout_passes=False, disable_bounds_checks=True),
)
def _retile_kernel(tphys, trm, b1a, b1b, b2a, b2b, semi, semo):
    w = _wid()
    iota = lax.iota(jnp.int32, 16)
    b1 = (b1a, b1b)
    b2 = (b2a, b2b)

    def in_dmas(k, par):
        rt = w * RT_PER_W + k
        return [
            pltpu.make_async_copy(
                tphys.at[pl.ds((ct * RT + rt) * 1024, 1024)],
                b1[par].at[pl.ds(ct * 1024, 1024)],
                semi.at[par],
            )
            for ct in range(CT)
        ]

    def out_dma(k, par):
        rt = w * RT_PER_W + k
        return pltpu.make_async_copy(
            b2[par], trm.at[pl.ds(rt * 4096, 4096)], semo.at[par]
        )

    for d in in_dmas(0, 0):
        d.start()
    for d in in_dmas(1, 1):
        d.start()

    def pair(k2, _):
        for par in range(2):
            k = k2 * 2 + par
            for d in in_dmas(k, par):
                d.wait()

            @pl.when(k2 >= 1)
            def _():
                out_dma(k - 2, par).wait()

            src, dst = b1[par], b2[par]

            @plsc.parallel_loop(0, 16, 1, unroll=2)
            def _(dk):
                csk = jnp.bitwise_and(iota + dk, 15)
                psrc = csk * 128 + iota
                qdst = iota * 32 + csk
                for c0 in (0, 16):
                    for r8 in range(8):
                        rl0 = r8 * 16
                        val = plsc.load_gather(
                            src, [psrc + (c0 * 128 + rl0)]
                        )
                        plsc.store_scatter(
                            dst, [qdst + (rl0 * 32 + c0)], val
                        )
            out_dma(k, par).start()

            @pl.when(k2 < RT_PER_W // 2 - 1)
            def _():
                for d in in_dmas(k + 2, par):
                    d.start()
        return ()

    lax.fori_loop(0, RT_PER_W // 2, pair, ())
    for par in range(2):
        out_dma(RT_PER_W - 2 + par, par).wait()


@functools.partial(
    pl.kernel,
    out_type=jax.ShapeDtypeStruct((B * EMB,), jnp.float32),
    mesh=_mesh,
    scratch_types=[
        pltpu.VMEM((1024,), jnp.int32),
        pltpu.VMEM((1024,), jnp.int32),
        pltpu.VMEM((1024, EMB), jnp.float32),
        pltpu.VMEM((1024, EMB), jnp.float32),
        pltpu.VMEM((2056,), jnp.float32),
        pltpu.VMEM((2056,), jnp.float32),
        pltpu.VMEM((2056,), jnp.float32),
        pltpu.VMEM((2056,), jnp.float32),
        pltpu.VMEM((2056,), jnp.float32),
        pltpu.VMEM((2056,), jnp.float32),
        pltpu.VMEM((2056,), jnp.float32),
        pltpu.VMEM((2056,), jnp.float32),
        pltpu.SemaphoreType.DMA((2,)),
        pltpu.SemaphoreType.DMA((2,)),
        pltpu.SemaphoreType.DMA((8,)),
    ],
    compiler_params=pltpu.CompilerParams(use_tc_tiling_on_sc=False, needs_layout_passes=False, disable_bounds_checks=True),
)
def _gather_kernel(cphys, trm, outh, idxa, idxb, ga, gbuf,
                   t0, t1, t2, t3, t4, t5, t6, t7, semi, semg, semo):
    tt = (t0, t1, t2, t3, t4, t5, t6, t7)
    w = _wid()
    iota = lax.iota(jnp.int32, 16)
    idxs = (idxa, idxb)
    gs = (ga, gbuf)

    def idx_dma(p, par):
        tile = w * TILE_PER_W + p
        return pltpu.make_async_copy(
            cphys.at[pl.ds(tile * 1024, 1024)], idxs[par], semi.at[par]
        )

    def gather_dma(par):
        return pltpu.make_async_copy(trm.at[idxs[par]], gs[par], semg.at[par])

    idx_dma(0, 0).start()
    idx_dma(0, 0).wait()
    gather_dma(0).start()
    idx_dma(1, 1).start()

    i7 = jnp.bitwise_and(iota, 7)
    h8 = lax.shift_right_logical(iota, 3) * 8
    hskew = lax.shift_right_logical(iota, 3) * 1032 + i7

    def emit(p, par):
        tile = w * TILE_PER_W + p
        ht = tile // BT
        bt = tile - ht * BT
        src = gs[par]

        def hl_body(hl2, _):
            for hlq in range(4):
                hl = hl2 * 4 + hlq
                for ehp in range(2):
                    slot = hlq * 2 + ehp

                    @pl.when((p > 0) | (hl2 > 0))
                    def _():
                        for _h in range(2):
                            pltpu.make_async_copy(
                                tt[slot].at[pl.ds(0, 1024)],
                                outh.at[pl.ds(0, 1024)],
                                semo.at[slot],
                            ).wait()

                    dst = tt[slot]

                    @plsc.parallel_loop(0, 8, 1, unroll=2)
                    def _(dk):
                        csk = jnp.bitwise_and(i7 + dk, 7)
                        cols = h8 + csk + ehp * 16
                        wdst = hskew + csk * 128
                        for b16 in range(16):
                            rows = i7 + (hl * 128 + b16 * 8)
                            val = plsc.load_gather(src, [rows, cols])
                            plsc.store_scatter(dst, [wdst + b16 * 8], val)
                    for h in range(2):
                        off = (
                            ((ht * 8 + hl) * 4 + ehp * 2 + h) * 128 + bt
                        ) * 1024
                        pltpu.async_copy(
                            dst.at[pl.ds(h * 1032, 1024)],
                            outh.at[pl.ds(off, 1024)],
                            semo.at[slot],
                        )
            return ()

        lax.fori_loop(0, 2, hl_body, ())

    def pair(p2, _):
        more = p2 < TILE_PER_W // 2 - 1
        p = 2 * p2

        idx_dma(p + 1, 1).wait()
        gather_dma(1).start()
        gather_dma(0).wait()

        @pl.when(more)
        def _():
            idx_dma(p + 2, 0).start()

        emit(p, 0)

        @pl.when(more)
        def _():
            idx_dma(p + 2, 0).wait()
            gather_dma(0).start()

        gather_dma(1).wait()

        @pl.when(more)
        def _():
            idx_dma(p + 3, 1).start()

        emit(p + 1, 1)
        return ()

    lax.fori_loop(0, TILE_PER_W // 2, pair, ())
    for slot in range(8):
        for _h in range(2):
            pltpu.make_async_copy(
                tt[slot].at[pl.ds(0, 1024)], outh.at[pl.ds(0, 1024)],
                semo.at[slot],
            ).wait()


@jax.jit
def kernel(condition, table):
    cphys = jnp.transpose(
        condition.reshape(BT, BL, HT, HL), (2, 0, 3, 1)
    ).reshape(-1)
    tphys = jnp.transpose(
        table.reshape(RT, RL, CT, CL), (2, 0, 3, 1)
    ).reshape(-1)
    trm = _retile_kernel(tphys).reshape(VOCAB, EMB)
    outflat = _gather_kernel(cphys, trm)
    out = jnp.transpose(
        outflat.reshape(HIST, 4, BT, 8, BL), (2, 4, 0, 1, 3)
    ).reshape(BATCH, HIST, EMB)
    return out


# speedup vs baseline: 2.5720x; 1.6478x over previous
import functools

import jax
import jax.numpy as jnp
from jax import lax
from jax.experimental import pallas as pl
from jax.experimental.pallas import tpu as pltpu
from jax.experimental.pallas import tpu_sc as plsc

NC, NS = 2, 16
NW = NC * NS
BATCH, HIST, EMB = 16384, 200, 32
B = BATCH * HIST
VOCAB = 1048576

CT, RT, CL, RL = 4, 8192, 8, 128
RT_PER_W = RT // NW
HT, BT, HL, BL = 25, 128, 8, 128
NTILE = HT * BT
TILE_PER_W = NTILE // NW

_mesh = plsc.VectorSubcoreMesh(core_axis_name="c", subcore_axis_name="s")


def _wid():
    return lax.axis_index("s") * NC + lax.axis_index("c")


@functools.partial(
    pl.kernel,
    out_type=jax.ShapeDtypeStruct((VOCAB * EMB,), jnp.float32),
    mesh=_mesh,
    scratch_types=[
        pltpu.VMEM((CT * 1024,), jnp.float32),
        pltpu.VMEM((CT * 1024,), jnp.float32),
        pltpu.VMEM((4096,), jnp.float32),
        pltpu.VMEM((4096,), jnp.float32),
        pltpu.SemaphoreType.DMA((2,)),
        pltpu.SemaphoreType.DMA((2,)),
    ],
    compiler_params=pltpu.CompilerParams(use_tc_tiling_on_sc=False, needs_layout_passes=False, disable_bounds_checks=True),
)
def _retile_kernel(tphys, trm, b1a, b1b, b2a, b2b, semi, semo):
    w = _wid()
    iota = lax.iota(jnp.int32, 16)
    b1 = (b1a, b1b)
    b2 = (b2a, b2b)

    def in_dmas(k, par):
        rt = w * RT_PER_W + k
        return [
            pltpu.make_async_copy(
                tphys.at[pl.ds((ct * RT + rt) * 1024, 1024)],
                b1[par].at[pl.ds(ct * 1024, 1024)],
                semi.at[par],
            )
            for ct in range(CT)
        ]

    def out_dma(k, par):
        rt = w * RT_PER_W + k
        return pltpu.make_async_copy(
            b2[par], trm.at[pl.ds(rt * 4096, 4096)], semo.at[par]
        )

    for d in in_dmas(0, 0):
        d.start()
    for d in in_dmas(1, 1):
        d.start()

    def pair(k2, _):
        for par in range(2):
            k = k2 * 2 + par
            for d in in_dmas(k, par):
                d.wait()

            @pl.when(k2 >= 1)
            def _():
                out_dma(k - 2, par).wait()

            src, dst = b1[par], b2[par]

            @plsc.parallel_loop(0, 16, 1, unroll=4)
            def _(dk):
                csk = jnp.bitwise_and(iota + dk, 15)
                psrc = csk * 128 + iota
                qdst = iota * 32 + csk
                for c0 in (0, 16):
                    for r8 in range(8):
                        rl0 = r8 * 16
                        val = plsc.load_gather(
                            src, [psrc + (c0 * 128 + rl0)]
                        )
                        plsc.store_scatter(
                            dst, [qdst + (rl0 * 32 + c0)], val
                        )
            out_dma(k, par).start()

            @pl.when(k2 < RT_PER_W // 2 - 1)
            def _():
                for d in in_dmas(k + 2, par):
                    d.start()
        return ()

    lax.fori_loop(0, RT_PER_W // 2, pair, ())
    for par in range(2):
        out_dma(RT_PER_W - 2 + par, par).wait()


@functools.partial(
    pl.kernel,
    out_type=jax.ShapeDtypeStruct((B * EMB,), jnp.float32),
    mesh=_mesh,
    scratch_types=[
        pltpu.VMEM((1024,), jnp.int32),
        pltpu.VMEM((1024,), jnp.int32),
        pltpu.VMEM((1024, EMB), jnp.float32),
        pltpu.VMEM((1024, EMB), jnp.float32),
        pltpu.VMEM((2056,), jnp.float32),
        pltpu.VMEM((2056,), jnp.float32),
        pltpu.VMEM((2056,), jnp.float32),
        pltpu.VMEM((2056,), jnp.float32),
        pltpu.VMEM((2056,), jnp.float32),
        pltpu.VMEM((2056,), jnp.float32),
        pltpu.VMEM((2056,), jnp.float32),
        pltpu.VMEM((2056,), jnp.float32),
        pltpu.SemaphoreType.DMA((2,)),
        pltpu.SemaphoreType.DMA((2,)),
        pltpu.SemaphoreType.DMA((8,)),
    ],
    compiler_params=pltpu.CompilerParams(use_tc_tiling_on_sc=False, needs_layout_passes=False, disable_bounds_checks=True),
)
def _gather_kernel(cphys, trm, outh, idxa, idxb, ga, gbuf,
                   t0, t1, t2, t3, t4, t5, t6, t7, semi, semg, semo):
    tt = (t0, t1, t2, t3, t4, t5, t6, t7)
    w = _wid()
    iota = lax.iota(jnp.int32, 16)
    idxs = (idxa, idxb)
    gs = (ga, gbuf)

    def idx_dma(p, par):
        tile = w * TILE_PER_W + p
        return pltpu.make_async_copy(
            cphys.at[pl.ds(tile * 1024, 1024)], idxs[par], semi.at[par]
        )

    def gather_dma(par):
        return pltpu.make_async_copy(trm.at[idxs[par]], gs[par], semg.at[par])

    idx_dma(0, 0).start()
    idx_dma(0, 0).wait()
    gather_dma(0).start()
    idx_dma(1, 1).start()

    i7 = jnp.bitwise_and(iota, 7)
    h8 = lax.shift_right_logical(iota, 3) * 8
    hskew = lax.shift_right_logical(iota, 3) * 1032 + i7

    def emit(p, par):
        tile = w * TILE_PER_W + p
        ht = tile // BT
        bt = tile - ht * BT
        src = gs[par]

        def hl_body(hl2, _):
            for hlq in range(4):
                hl = hl2 * 4 + hlq
                for ehp in range(2):
                    slot = hlq * 2 + ehp

                    @pl.when((p > 0) | (hl2 > 0))
                    def _():
                        for _h in range(2):
                            pltpu.make_async_copy(
                                tt[slot].at[pl.ds(0, 1024)],
                                outh.at[pl.ds(0, 1024)],
                                semo.at[slot],
                            ).wait()

                    dst = tt[slot]

                    @plsc.parallel_loop(0, 8, 1, unroll=4)
                    def _(dk):
                        csk = jnp.bitwise_and(i7 + dk, 7)
                        cols = h8 + csk + ehp * 16
                        wdst = hskew + csk * 128
                        for b16 in range(16):
                            rows = i7 + (hl * 128 + b16 * 8)
                            val = plsc.load_gather(src, [rows, cols])
                            plsc.store_scatter(dst, [wdst + b16 * 8], val)
                    for h in range(2):
                        off = (
                            ((ht * 8 + hl) * 4 + ehp * 2 + h) * 128 + bt
                        ) * 1024
                        pltpu.async_copy(
                            dst.at[pl.ds(h * 1032, 1024)],
                            outh.at[pl.ds(off, 1024)],
                            semo.at[slot],
                        )
            return ()

        lax.fori_loop(0, 2, hl_body, ())

    def pair(p2, _):
        more = p2 < TILE_PER_W // 2 - 1
        p = 2 * p2

        idx_dma(p + 1, 1).wait()
        gather_dma(1).start()
        gather_dma(0).wait()

        @pl.when(more)
        def _():
            idx_dma(p + 2, 0).start()

        emit(p, 0)

        @pl.when(more)
        def _():
            idx_dma(p + 2, 0).wait()
            gather_dma(0).start()

        gather_dma(1).wait()

        @pl.when(more)
        def _():
            idx_dma(p + 3, 1).start()

        emit(p + 1, 1)
        return ()

    lax.fori_loop(0, TILE_PER_W // 2, pair, ())
    for slot in range(8):
        for _h in range(2):
            pltpu.make_async_copy(
                tt[slot].at[pl.ds(0, 1024)], outh.at[pl.ds(0, 1024)],
                semo.at[slot],
            ).wait()


@jax.jit
def kernel(condition, table):
    cphys = jnp.transpose(
        condition.reshape(BT, BL, HT, HL), (2, 0, 3, 1)
    ).reshape(-1)
    tphys = jnp.transpose(
        table.reshape(RT, RL, CT, CL), (2, 0, 3, 1)
    ).reshape(-1)
    trm = _retile_kernel(tphys).reshape(VOCAB, EMB)
    outflat = _gather_kernel(cphys, trm)
    out = jnp.transpose(
        outflat.reshape(HIST, 4, BT, 8, BL), (2, 4, 0, 1, 3)
    ).reshape(BATCH, HIST, EMB)
    return out
